# baseline (device time: 134423 ns/iter reference)
import jax
import jax.numpy as jnp
from jax import lax
from jax.experimental import pallas as pl
from jax.experimental.pallas import tpu as pltpu

N_DEV = 4


def kernel(x, w_mat):
    m_per, k = x.shape
    _, n_per = w_mat.shape
    kh = k // 2
    mh = m_per // 2

    xb = x.astype(jnp.bfloat16)
    wb = w_mat.astype(jnp.bfloat16)

    def body(x_hbm, w_ref, out_ref, wg, xbf, yg, ysnd, amax_ref,
             wssem, wrsem, yssem, yrsem, asend, arecv, lsem):
        my = lax.axis_index("i")
        left = lax.rem(my + N_DEV - 1, N_DEV)
        right = lax.rem(my + 1, N_DEV)
        diag = lax.rem(my + 2, N_DEV)
        MESH = pl.DeviceIdType.MESH

        bsem = pltpu.get_barrier_semaphore()
        for nbr in (left, right, diag):
            pl.semaphore_signal(bsem, inc=1, device_id=(nbr,),
                                device_id_type=MESH)
        pl.semaphore_wait(bsem, 3)

        h1r = pltpu.make_async_remote_copy(
            src_ref=w_ref, dst_ref=wg.at[my],
            send_sem=wssem.at[0], recv_sem=wrsem.at[0],
            device_id=(right,), device_id_type=MESH)
        h1l = pltpu.make_async_remote_copy(
            src_ref=w_ref, dst_ref=wg.at[my],
            send_sem=wssem.at[1], recv_sem=wrsem.at[1],
            device_id=(left,), device_id_type=MESH)
        h1r.start()
        h1l.start()

        cpx = pltpu.make_async_copy(x_hbm, xbf, lsem)
        cpx.start()
        cpx.wait()

        def mm_block(wv):
            return [
                lax.dot_general(
                    xbf[pl.ds(t * mh, mh), :], wv,
                    (((1,), (0,)), ((), ())),
                    preferred_element_type=jnp.float32)
                for t in range(2)
            ]

        am = jnp.float32(0.0)
        for t, yt in enumerate(mm_block(w_ref[...])):
            am = jnp.maximum(am, jnp.max(jnp.abs(yt)))
            yg[my, pl.ds(t * mh, mh), :] = yt.astype(jnp.bfloat16)

        h1r.wait_recv()
        h2r = pltpu.make_async_remote_copy(
            src_ref=wg.at[left, pl.ds(0, kh)],
            dst_ref=wg.at[left, pl.ds(0, kh)],
            send_sem=wssem.at[2], recv_sem=wrsem.at[2],
            device_id=(right,), device_id_type=MESH)
        h2r.start()
        h1l.wait_recv()
        h2l = pltpu.make_async_remote_copy(
            src_ref=wg.at[right, pl.ds(kh, kh)],
            dst_ref=wg.at[right, pl.ds(kh, kh)],
            send_sem=wssem.at[3], recv_sem=wrsem.at[3],
            device_id=(left,), device_id_type=MESH)
        h2l.start()

        for t, yt in enumerate(mm_block(wg[left])):
            am = jnp.maximum(am, jnp.max(jnp.abs(yt)))
            ysnd[0, pl.ds(t * mh, mh), :] = yt.astype(jnp.bfloat16)
        ysl = pltpu.make_async_remote_copy(
            src_ref=ysnd.at[0], dst_ref=yg.at[my],
            send_sem=yssem.at[0], recv_sem=yrsem.at[my],
            device_id=(left,), device_id_type=MESH)
        ysl.start()
        for t, yt in enumerate(mm_block(wg[right])):
            am = jnp.maximum(am, jnp.max(jnp.abs(yt)))
            ysnd[1, pl.ds(t * mh, mh), :] = yt.astype(jnp.bfloat16)
        ysr = pltpu.make_async_remote_copy(
            src_ref=ysnd.at[1], dst_ref=yg.at[my],
            send_sem=yssem.at[1], recv_sem=yrsem.at[my],
            device_id=(right,), device_id_type=MESH)
        ysr.start()

        h2r.wait_recv()
        h2l.wait_recv()
        for t, yt in enumerate(mm_block(wg[diag])):
            am = jnp.maximum(am, jnp.max(jnp.abs(yt)))
            ysnd[2, pl.ds(t * mh, mh), :] = yt.astype(jnp.bfloat16)
        ysd = pltpu.make_async_remote_copy(
            src_ref=ysnd.at[2], dst_ref=yg.at[my],
            send_sem=yssem.at[2], recv_sem=yrsem.at[my],
            device_id=(diag,), device_id_type=MESH)
        ysd.start()

        amax_ref[my] = jnp.full((8, 128), am, jnp.float32)
        asends = []
        for j, tgt in enumerate((left, right, diag)):
            s = pltpu.make_async_remote_copy(
                src_ref=amax_ref.at[my], dst_ref=amax_ref.at[my],
                send_sem=asend.at[j], recv_sem=arecv.at[my],
                device_id=(tgt,), device_id_type=MESH)
            s.start()
            asends.append(s)

        for s in (left, right, diag):
            rv = pltpu.make_async_remote_copy(
                src_ref=yg.at[s], dst_ref=yg.at[s],
                send_sem=yssem.at[3], recv_sem=yrsem.at[s],
                device_id=(s,), device_id_type=MESH)
            rv.wait_recv()
        for src_dev in (left, right, diag):
            rv = pltpu.make_async_remote_copy(
                src_ref=amax_ref.at[src_dev], dst_ref=amax_ref.at[src_dev],
                send_sem=asend.at[3], recv_sem=arecv.at[src_dev],
                device_id=(src_dev,), device_id_type=MESH)
            rv.wait_recv()

        g_amax = jnp.max(amax_ref[...])
        scale = g_amax / 127.0
        for j in range(N_DEV):
            blk = yg[j].astype(jnp.float32)
            q = jnp.clip(jnp.round(blk / scale), -127.0, 127.0)
            out_ref[pl.ds(j * m_per, m_per), :] = q * scale

        for d in (h1r, h1l, h2r, h2l, ysl, ysr, ysd, *asends):
            d.wait_send()

    out, _, _ = pl.pallas_call(
        body,
        out_shape=(
            jax.ShapeDtypeStruct((N_DEV * m_per, n_per), jnp.float32),
            jax.ShapeDtypeStruct((N_DEV, k, n_per), jnp.bfloat16),
            jax.ShapeDtypeStruct((m_per, k), jnp.bfloat16),
        ),
        in_specs=[
            pl.BlockSpec(memory_space=pl.ANY),
            pl.BlockSpec(memory_space=pltpu.VMEM),
        ],
        out_specs=(
            pl.BlockSpec(memory_space=pltpu.VMEM),
            pl.BlockSpec(memory_space=pltpu.VMEM),
            pl.BlockSpec(memory_space=pltpu.VMEM),
        ),
        scratch_shapes=[
            pltpu.VMEM((N_DEV, m_per, n_per), jnp.bfloat16),
            pltpu.VMEM((3, m_per, n_per), jnp.bfloat16),
            pltpu.VMEM((N_DEV, 8, 128), jnp.float32),
            pltpu.SemaphoreType.DMA((4,)),
            pltpu.SemaphoreType.DMA((4,)),
            pltpu.SemaphoreType.DMA((4,)),
            pltpu.SemaphoreType.DMA((4,)),
            pltpu.SemaphoreType.DMA((4,)),
            pltpu.SemaphoreType.DMA((4,)),
            pltpu.SemaphoreType.DMA(()),
        ],
        compiler_params=pltpu.CompilerParams(
            collective_id=0, vmem_limit_bytes=100 * 1024 * 1024),
    )(xb, wb)
    return out


# device time: 120528 ns/iter; 1.1153x vs baseline; 1.1153x over previous
import jax
import jax.numpy as jnp
from jax import lax
from jax.experimental import pallas as pl
from jax.experimental.pallas import tpu as pltpu

N_DEV = 4


def kernel(x, w_mat):
    m_per, k = x.shape
    _, n_per = w_mat.shape
    kh = k // 2
    mh = m_per // 2

    def body(x_hbm, w_ref, out_ref, wg, xbf, stage, yg, ysnd, amax_ref,
             wssem, wrsem, yssem, yrsem, asend, arecv, lsem):
        my = lax.axis_index("i")
        left = lax.rem(my + N_DEV - 1, N_DEV)
        right = lax.rem(my + 1, N_DEV)
        diag = lax.rem(my + 2, N_DEV)
        MESH = pl.DeviceIdType.MESH

        bsem = pltpu.get_barrier_semaphore()
        for nbr in (left, right, diag):
            pl.semaphore_signal(bsem, inc=1, device_id=(nbr,),
                                device_id_type=MESH)
        pl.semaphore_wait(bsem, 3)

        wg[my] = w_ref[...].astype(jnp.bfloat16)
        h1r = pltpu.make_async_remote_copy(
            src_ref=wg.at[my], dst_ref=wg.at[my],
            send_sem=wssem.at[0], recv_sem=wrsem.at[0],
            device_id=(right,), device_id_type=MESH)
        h1l = pltpu.make_async_remote_copy(
            src_ref=wg.at[my], dst_ref=wg.at[my],
            send_sem=wssem.at[1], recv_sem=wrsem.at[1],
            device_id=(left,), device_id_type=MESH)
        h1r.start()
        h1l.start()

        for t in range(2):
            cpx = pltpu.make_async_copy(
                x_hbm.at[pl.ds(t * mh, mh)], stage, lsem)
            cpx.start()
            cpx.wait()
            xbf[pl.ds(t * mh, mh), :] = stage[...].astype(jnp.bfloat16)

        def mm_block(wv):
            return [
                lax.dot_general(
                    xbf[pl.ds(t * mh, mh), :], wv,
                    (((1,), (0,)), ((), ())),
                    preferred_element_type=jnp.float32)
                for t in range(2)
            ]

        am = jnp.float32(0.0)
        for t, yt in enumerate(mm_block(wg[my])):
            am = jnp.maximum(am, jnp.max(jnp.abs(yt)))
            yg[my, pl.ds(t * mh, mh), :] = yt.astype(jnp.bfloat16)

        h1r.wait_recv()
        h2r = pltpu.make_async_remote_copy(
            src_ref=wg.at[left, pl.ds(0, kh)],
            dst_ref=wg.at[left, pl.ds(0, kh)],
            send_sem=wssem.at[2], recv_sem=wrsem.at[2],
            device_id=(right,), device_id_type=MESH)
        h2r.start()
        h1l.wait_recv()
        h2l = pltpu.make_async_remote_copy(
            src_ref=wg.at[right, pl.ds(kh, kh)],
            dst_ref=wg.at[right, pl.ds(kh, kh)],
            send_sem=wssem.at[3], recv_sem=wrsem.at[3],
            device_id=(left,), device_id_type=MESH)
        h2l.start()

        for t, yt in enumerate(mm_block(wg[left])):
            am = jnp.maximum(am, jnp.max(jnp.abs(yt)))
            ysnd[0, pl.ds(t * mh, mh), :] = yt.astype(jnp.bfloat16)
        ysl = pltpu.make_async_remote_copy(
            src_ref=ysnd.at[0], dst_ref=yg.at[my],
            send_sem=yssem.at[0], recv_sem=yrsem.at[my],
            device_id=(left,), device_id_type=MESH)
        ysl.start()
        for t, yt in enumerate(mm_block(wg[right])):
            am = jnp.maximum(am, jnp.max(jnp.abs(yt)))
            ysnd[1, pl.ds(t * mh, mh), :] = yt.astype(jnp.bfloat16)
        ysr = pltpu.make_async_remote_copy(
            src_ref=ysnd.at[1], dst_ref=yg.at[my],
            send_sem=yssem.at[1], recv_sem=yrsem.at[my],
            device_id=(right,), device_id_type=MESH)
        ysr.start()

        h2r.wait_recv()
        h2l.wait_recv()
        for t, yt in enumerate(mm_block(wg[diag])):
            am = jnp.maximum(am, jnp.max(jnp.abs(yt)))
            ysnd[2, pl.ds(t * mh, mh), :] = yt.astype(jnp.bfloat16)
        ysd = pltpu.make_async_remote_copy(
            src_ref=ysnd.at[2], dst_ref=yg.at[my],
            send_sem=yssem.at[2], recv_sem=yrsem.at[my],
            device_id=(diag,), device_id_type=MESH)
        ysd.start()

        amax_ref[my] = jnp.full((8, 128), am, jnp.float32)
        asends = []
        for j, tgt in enumerate((left, right, diag)):
            s = pltpu.make_async_remote_copy(
                src_ref=amax_ref.at[my], dst_ref=amax_ref.at[my],
                send_sem=asend.at[j], recv_sem=arecv.at[my],
                device_id=(tgt,), device_id_type=MESH)
            s.start()
            asends.append(s)

        for s in (left, right, diag):
            rv = pltpu.make_async_remote_copy(
                src_ref=yg.at[s], dst_ref=yg.at[s],
                send_sem=yssem.at[3], recv_sem=yrsem.at[s],
                device_id=(s,), device_id_type=MESH)
            rv.wait_recv()
        for src_dev in (left, right, diag):
            rv = pltpu.make_async_remote_copy(
                src_ref=amax_ref.at[src_dev], dst_ref=amax_ref.at[src_dev],
                send_sem=asend.at[3], recv_sem=arecv.at[src_dev],
                device_id=(src_dev,), device_id_type=MESH)
            rv.wait_recv()

        g_amax = jnp.max(amax_ref[...])
        scale = g_amax / 127.0
        for j in range(N_DEV):
            blk = yg[j].astype(jnp.float32)
            q = jnp.clip(jnp.round(blk / scale), -127.0, 127.0)
            out_ref[pl.ds(j * m_per, m_per), :] = q * scale

        for d in (h1r, h1l, h2r, h2l, ysl, ysr, ysd, *asends):
            d.wait_send()

    out, _, _ = pl.pallas_call(
        body,
        out_shape=(
            jax.ShapeDtypeStruct((N_DEV * m_per, n_per), jnp.float32),
            jax.ShapeDtypeStruct((N_DEV, k, n_per), jnp.bfloat16),
            jax.ShapeDtypeStruct((m_per, k), jnp.bfloat16),
        ),
        in_specs=[
            pl.BlockSpec(memory_space=pl.ANY),
            pl.BlockSpec(memory_space=pltpu.VMEM),
        ],
        out_specs=(
            pl.BlockSpec(memory_space=pltpu.VMEM),
            pl.BlockSpec(memory_space=pltpu.VMEM),
            pl.BlockSpec(memory_space=pltpu.VMEM),
        ),
        scratch_shapes=[
            pltpu.VMEM((mh, k), jnp.float32),
            pltpu.VMEM((N_DEV, m_per, n_per), jnp.bfloat16),
            pltpu.VMEM((3, m_per, n_per), jnp.bfloat16),
            pltpu.VMEM((N_DEV, 8, 128), jnp.float32),
            pltpu.SemaphoreType.DMA((4,)),
            pltpu.SemaphoreType.DMA((4,)),
            pltpu.SemaphoreType.DMA((4,)),
            pltpu.SemaphoreType.DMA((4,)),
            pltpu.SemaphoreType.DMA((4,)),
            pltpu.SemaphoreType.DMA((4,)),
            pltpu.SemaphoreType.DMA(()),
        ],
        compiler_params=pltpu.CompilerParams(
            collective_id=0, vmem_limit_bytes=100 * 1024 * 1024),
    )(x, w_mat)
    return out


# device time: 120388 ns/iter; 1.1166x vs baseline; 1.0012x over previous
import jax
import jax.numpy as jnp
from jax import lax
from jax.experimental import pallas as pl
from jax.experimental.pallas import tpu as pltpu

N_DEV = 4


def kernel(x, w_mat):
    m_per, k = x.shape
    _, n_per = w_mat.shape
    kh = k // 2
    mh = m_per // 2

    def body(x_hbm, w_ref, out_ref, wg, xbf, stage, yg, ysnd, amax_ref,
             wssem, wrsem, yssem, yrsem, asend, arecv, lsem):
        my = lax.axis_index("i")
        left = lax.rem(my + N_DEV - 1, N_DEV)
        right = lax.rem(my + 1, N_DEV)
        diag = lax.rem(my + 2, N_DEV)
        MESH = pl.DeviceIdType.MESH

        bsem = pltpu.get_barrier_semaphore()
        for nbr in (left, right, diag):
            pl.semaphore_signal(bsem, inc=1, device_id=(nbr,),
                                device_id_type=MESH)
        pl.semaphore_wait(bsem, 3)

        wg[my] = w_ref[...].astype(jnp.bfloat16)
        h1r = pltpu.make_async_remote_copy(
            src_ref=wg.at[my], dst_ref=wg.at[my],
            send_sem=wssem.at[0], recv_sem=wrsem.at[0],
            device_id=(right,), device_id_type=MESH)
        h1l = pltpu.make_async_remote_copy(
            src_ref=wg.at[my], dst_ref=wg.at[my],
            send_sem=wssem.at[1], recv_sem=wrsem.at[1],
            device_id=(left,), device_id_type=MESH)
        h1r.start()
        h1l.start()

        for t in range(2):
            cpx = pltpu.make_async_copy(
                x_hbm.at[pl.ds(t * mh, mh)], stage, lsem)
            cpx.start()
            cpx.wait()
            xbf[pl.ds(t * mh, mh), :] = stage[...].astype(jnp.bfloat16)

        def mm_block(wv):
            return [
                lax.dot_general(
                    xbf[pl.ds(t * mh, mh), :], wv,
                    (((1,), (0,)), ((), ())),
                    preferred_element_type=jnp.float32)
                for t in range(2)
            ]

        am = jnp.float32(0.0)
        for t, yt in enumerate(mm_block(wg[my])):
            am = jnp.maximum(am, jnp.max(jnp.abs(yt)))
            yg[my, pl.ds(t * mh, mh), :] = yt.astype(jnp.bfloat16)

        h1r.wait_recv()
        h2r = pltpu.make_async_remote_copy(
            src_ref=wg.at[left, pl.ds(0, kh)],
            dst_ref=wg.at[left, pl.ds(0, kh)],
            send_sem=wssem.at[2], recv_sem=wrsem.at[2],
            device_id=(right,), device_id_type=MESH)
        h2r.start()
        h1l.wait_recv()
        h2l = pltpu.make_async_remote_copy(
            src_ref=wg.at[right, pl.ds(kh, kh)],
            dst_ref=wg.at[right, pl.ds(kh, kh)],
            send_sem=wssem.at[3], recv_sem=wrsem.at[3],
            device_id=(left,), device_id_type=MESH)
        h2l.start()

        for t, yt in enumerate(mm_block(wg[left])):
            am = jnp.maximum(am, jnp.max(jnp.abs(yt)))
            ysnd[0, pl.ds(t * mh, mh), :] = yt.astype(jnp.bfloat16)
        ysl = pltpu.make_async_remote_copy(
            src_ref=ysnd.at[0], dst_ref=yg.at[my],
            send_sem=yssem.at[0], recv_sem=yrsem.at[my],
            device_id=(left,), device_id_type=MESH)
        ysl.start()
        for t, yt in enumerate(mm_block(wg[right])):
            am = jnp.maximum(am, jnp.max(jnp.abs(yt)))
            ysnd[1, pl.ds(t * mh, mh), :] = yt.astype(jnp.bfloat16)
        ysr = pltpu.make_async_remote_copy(
            src_ref=ysnd.at[1], dst_ref=yg.at[my],
            send_sem=yssem.at[1], recv_sem=yrsem.at[my],
            device_id=(right,), device_id_type=MESH)
        ysr.start()

        h2r.wait_recv()
        h2l.wait_recv()
        for t, yt in enumerate(mm_block(wg[diag])):
            am = jnp.maximum(am, jnp.max(jnp.abs(yt)))
            ysnd[2, pl.ds(t * mh, mh), :] = yt.astype(jnp.bfloat16)
        ysd = pltpu.make_async_remote_copy(
            src_ref=ysnd.at[2], dst_ref=yg.at[my],
            send_sem=yssem.at[2], recv_sem=yrsem.at[my],
            device_id=(diag,), device_id_type=MESH)
        ysd.start()

        amax_ref[my] = jnp.full((8, 128), am, jnp.float32)
        asends = []
        for j, tgt in enumerate((left, right, diag)):
            s = pltpu.make_async_remote_copy(
                src_ref=amax_ref.at[my], dst_ref=amax_ref.at[my],
                send_sem=asend.at[j], recv_sem=arecv.at[my],
                device_id=(tgt,), device_id_type=MESH)
            s.start()
            asends.append(s)

        for src_dev in (left, right, diag):
            rv = pltpu.make_async_remote_copy(
                src_ref=amax_ref.at[src_dev], dst_ref=amax_ref.at[src_dev],
                send_sem=asend.at[3], recv_sem=arecv.at[src_dev],
                device_id=(src_dev,), device_id_type=MESH)
            rv.wait_recv()
        g_amax = jnp.max(amax_ref[...])
        scale = g_amax / 127.0

        def qdq_store(slot):
            blk = yg[slot].astype(jnp.float32)
            q = jnp.clip(jnp.round(blk / scale), -127.0, 127.0)
            out_ref[pl.ds(slot * m_per, m_per), :] = q * scale

        qdq_store(my)
        for s in (left, right, diag):
            rv = pltpu.make_async_remote_copy(
                src_ref=yg.at[s], dst_ref=yg.at[s],
                send_sem=yssem.at[3], recv_sem=yrsem.at[s],
                device_id=(s,), device_id_type=MESH)
            rv.wait_recv()
            qdq_store(s)

        for d in (h1r, h1l, h2r, h2l, ysl, ysr, ysd, *asends):
            d.wait_send()

    out, _, _ = pl.pallas_call(
        body,
        out_shape=(
            jax.ShapeDtypeStruct((N_DEV * m_per, n_per), jnp.float32),
            jax.ShapeDtypeStruct((N_DEV, k, n_per), jnp.bfloat16),
            jax.ShapeDtypeStruct((m_per, k), jnp.bfloat16),
        ),
        in_specs=[
            pl.BlockSpec(memory_space=pl.ANY),
            pl.BlockSpec(memory_space=pltpu.VMEM),
        ],
        out_specs=(
            pl.BlockSpec(memory_space=pltpu.VMEM),
            pl.BlockSpec(memory_space=pltpu.VMEM),
            pl.BlockSpec(memory_space=pltpu.VMEM),
        ),
        scratch_shapes=[
            pltpu.VMEM((mh, k), jnp.float32),
            pltpu.VMEM((N_DEV, m_per, n_per), jnp.bfloat16),
            pltpu.VMEM((3, m_per, n_per), jnp.bfloat16),
            pltpu.VMEM((N_DEV, 8, 128), jnp.float32),
            pltpu.SemaphoreType.DMA((4,)),
            pltpu.SemaphoreType.DMA((4,)),
            pltpu.SemaphoreType.DMA((4,)),
            pltpu.SemaphoreType.DMA((4,)),
            pltpu.SemaphoreType.DMA((4,)),
            pltpu.SemaphoreType.DMA((4,)),
            pltpu.SemaphoreType.DMA(()),
        ],
        compiler_params=pltpu.CompilerParams(
            collective_id=0, vmem_limit_bytes=100 * 1024 * 1024),
    )(x, w_mat)
    return out
